# initial kernel scaffold (unmeasured)
import jax
import jax.numpy as jnp
from jax import lax
from jax.experimental import pallas as pl
from jax.experimental.pallas import tpu as pltpu


def kernel(
    x,
):
    def body(*refs):
        pass

    out_shape = jax.ShapeDtypeStruct(..., jnp.float32)
    return pl.pallas_call(body, out_shape=out_shape)(...)


# baseline (device time: 5937713 ns/iter reference)
import jax
import jax.numpy as jnp
from jax import lax
from jax.experimental import pallas as pl
from jax.experimental.pallas import tpu as pltpu

try:
    jax.config.update(
        "jax_compilation_cache_dir", "/tmp/scband_jax_compile_cache"
    )
    jax.config.update("jax_persistent_cache_min_compile_time_secs", 0.0)
    jax.config.update("jax_persistent_cache_min_entry_size_bytes", 0)
except Exception:
    pass

N_DEV = 4
LOCAL_CHUNK = 128
MERGE_CHUNK = 128


def _stage_dir(v, iota, log_j, asc):
    m = v.shape[0]
    j = jnp.int32(1) << log_j
    down = pltpu.roll(v, m - j, axis=0)
    up = pltpu.roll(v, j, axis=0)
    is_low = (iota & j) == 0
    partner = jnp.where(is_low, down, up)
    keep_min = is_low == asc
    return jnp.where(keep_min, jnp.minimum(v, partner), jnp.maximum(v, partner))


def _local_sort(v, desc):
    m = v.shape[0]
    iota = lax.broadcasted_iota(jnp.int32, (m, 1), 0)

    def phase_body(p, v):
        k = jnp.int32(1) << p
        asc = ((iota & k) == 0) != desc

        def step_body(s, v):
            return _stage_dir(v, iota, p - s - 1, asc)

        return lax.fori_loop(0, p, step_body, v)

    return lax.fori_loop(1, m.bit_length(), phase_body, v)


def _merge_half(w, asc):
    m = w.shape[0]
    n_st = m.bit_length() - 1
    iota = lax.broadcasted_iota(jnp.int32, (m, 1), 0)

    def step_body(s, w):
        return _stage_dir(w, iota, n_st - 1 - s, asc)

    return lax.fori_loop(0, n_st, step_body, w)


def kernel(x):
    m_per, n = x.shape
    assert n % LOCAL_CHUNK == 0 and n % MERGE_CHUNK == 0
    m_tot = N_DEV * m_per
    half = m_tot // 2

    def body(x_ref, out_ref, gath_ref, stage_ref, copy_sem, send_sems, recv_sems):
        my_pos = lax.axis_index("i")
        right = lax.rem(my_pos + 1, N_DEV)
        left = lax.rem(my_pos + N_DEV - 1, N_DEV)
        desc = (my_pos & 1) == 1
        my_rows = pl.ds(my_pos * m_per, m_per)

        def lchunk_body(ci, carry):
            cs = pl.ds(ci * LOCAL_CHUNK, LOCAL_CHUNK)
            out_ref[:, cs] = _local_sort(x_ref[:, cs], desc)
            return carry

        lax.fori_loop(0, n // LOCAL_CHUNK, lchunk_body, 0)
        cp = pltpu.make_async_copy(out_ref, gath_ref.at[my_rows, :], copy_sem)
        cp.start()
        cp.wait()

        barrier_sem = pltpu.get_barrier_semaphore()
        for nbr in (left, right):
            pl.semaphore_signal(
                barrier_sem, inc=1,
                device_id=(nbr,), device_id_type=pl.DeviceIdType.MESH,
            )
        pl.semaphore_wait(barrier_sem, 2)

        for h in range(N_DEV - 1):
            origin = lax.rem(my_pos - h + N_DEV, N_DEV)
            sl = pl.ds(origin * m_per, m_per)
            rdma = pltpu.make_async_remote_copy(
                src_ref=gath_ref.at[sl, :],
                dst_ref=gath_ref.at[sl, :],
                send_sem=send_sems.at[h],
                recv_sem=recv_sems.at[h],
                device_id=(right,),
                device_id_type=pl.DeviceIdType.MESH,
            )
            rdma.start()
            rdma.wait()

        def mchunk_body(ci, carry):
            cs = pl.ds(ci * MERGE_CHUNK, MERGE_CHUNK)
            cp = pltpu.make_async_copy(gath_ref.at[:, cs], stage_ref, copy_sem)
            cp.start()
            cp.wait()

            def phase_a(hb, carry):
                rows = pl.ds(hb * half, half)
                stage_ref[rows, :] = _merge_half(stage_ref[rows, :], hb == 0)
                return carry

            lax.fori_loop(0, 2, phase_a, 0)

            a = stage_ref[0:half, :]
            b = stage_ref[half:m_tot, :]
            stage_ref[0:half, :] = jnp.minimum(a, b)
            stage_ref[half:m_tot, :] = jnp.maximum(a, b)

            def phase_b(hb, carry):
                rows = pl.ds(hb * half, half)
                stage_ref[rows, :] = _merge_half(stage_ref[rows, :], True)
                return carry

            lax.fori_loop(0, 2, phase_b, 0)

            out_ref[:, cs] = stage_ref[my_rows, :]
            return carry

        lax.fori_loop(0, n // MERGE_CHUNK, mchunk_body, 0)

    out, _ = pl.pallas_call(
        body,
        out_shape=[
            jax.ShapeDtypeStruct((m_per, n), jnp.bfloat16),
            jax.ShapeDtypeStruct((m_tot, n), jnp.bfloat16),
        ],
        in_specs=[pl.BlockSpec(memory_space=pltpu.VMEM)],
        out_specs=[
            pl.BlockSpec(memory_space=pltpu.VMEM),
            pl.BlockSpec(memory_space=pltpu.MemorySpace.HBM),
        ],
        scratch_shapes=[
            pltpu.VMEM((m_tot, MERGE_CHUNK), jnp.bfloat16),
            pltpu.SemaphoreType.DMA,
            pltpu.SemaphoreType.DMA((N_DEV - 1,)),
            pltpu.SemaphoreType.DMA((N_DEV - 1,)),
        ],
        compiler_params=pltpu.CompilerParams(
            collective_id=0,
            vmem_limit_bytes=60 * 1024 * 1024,
        ),
    )(x.astype(jnp.bfloat16))
    return out
